# baseline (device time: 16443 ns/iter reference)
import jax
import jax.numpy as jnp
from jax import lax
from jax.experimental import pallas as pl
from jax.experimental.pallas import tpu as pltpu

N_DEV = 4
B, SQ, SKV, HQ_LOC, DH = 2, 256, 256, 4, 64
D_MODEL = 512
D_HEADS = HQ_LOC * DH
WINDOW = 128


def kernel(x, Wq, K_ext, V_ext, Wo):
    xf = x.reshape(B * SQ, D_MODEL)
    kf = K_ext.reshape(B, SKV, D_HEADS)
    vf = V_ext.reshape(B, SKV, D_HEADS)
    my_out = lax.axis_index("i")
    wq_dev = lax.dynamic_slice(Wq, (0, my_out * D_HEADS), (D_MODEL, D_HEADS))

    def body(x_ref, wq_ref, k_ref, v_ref, wo_ref, out_ref,
             xv, wqv, kv, vv, wov, ctx_all, copy_sems, send_sems, recv_sems):
        my = lax.axis_index("i")
        peers = [lax.rem(my + d, N_DEV) for d in (1, 3, 2)]
        col0 = my * D_HEADS

        cp_x = pltpu.make_async_copy(x_ref, xv, copy_sems.at[0])
        cp_wq = pltpu.make_async_copy(wq_ref, wqv, copy_sems.at[1])
        cp_k = pltpu.make_async_copy(k_ref, kv, copy_sems.at[2])
        cp_v = pltpu.make_async_copy(v_ref, vv, copy_sems.at[3])
        cp_wo = pltpu.make_async_copy(wo_ref, wov, copy_sems.at[4])
        for cp in (cp_x, cp_wq, cp_k, cp_v):
            cp.start()

        barrier_sem = pltpu.get_barrier_semaphore()
        for p in peers:
            pl.semaphore_signal(
                barrier_sem, inc=1,
                device_id=(p,), device_id_type=pl.DeviceIdType.MESH,
            )
        pl.semaphore_wait(barrier_sem, N_DEV - 1)

        cp_x.wait()
        cp_wq.wait()
        qf = jnp.dot((xv[:, :] * 0.125).astype(jnp.bfloat16),
                     wqv[:, :].astype(jnp.bfloat16),
                     preferred_element_type=jnp.float32,
                     ).astype(jnp.bfloat16)

        HALF = SQ // 2
        masks = []
        for j in range(2):
            qi = lax.rem(
                lax.broadcasted_iota(jnp.int32, (HQ_LOC * HALF, SKV), 0),
                HALF) + j * HALF
            ki = lax.broadcasted_iota(jnp.int32, (HQ_LOC * HALF, SKV), 1)
            masks.append(jnp.abs(qi - ki) <= WINDOW)

        cp_k.wait()
        cp_v.wait()
        cp_wo.start()
        sends = []
        for b in range(B):
            kb = kv[b, :, :].astype(jnp.bfloat16)
            vb = vv[b, :, :].astype(jnp.bfloat16)
            for j in range(2):
                r0 = b * SQ + j * HALF
                s_c = jnp.concatenate([
                    lax.dot_general(
                        qf[r0:r0 + HALF, h * DH:(h + 1) * DH],
                        kb[:, h * DH:(h + 1) * DH],
                        (((1,), (1,)), ((), ())),
                        preferred_element_type=jnp.float32,
                    ) for h in range(HQ_LOC)
                ], axis=0).astype(jnp.bfloat16)
                w_c = jnp.where(masks[j], jnp.exp(s_c),
                                jnp.bfloat16(0.0))
                denom = jnp.sum(w_c, axis=-1, keepdims=True,
                                dtype=jnp.float32)
                o_c = jnp.concatenate([
                    jnp.dot(w_c[h * HALF:(h + 1) * HALF, :],
                            vb[:, h * DH:(h + 1) * DH],
                            preferred_element_type=jnp.float32)
                    for h in range(HQ_LOC)
                ], axis=0)
                ctx_c = (o_c / denom).astype(jnp.bfloat16)
                for h in range(HQ_LOC):
                    ctx_all[my, b, j * HALF:(j + 1) * HALF,
                            h * DH:(h + 1) * DH] = (
                        ctx_c[h * HALF:(h + 1) * HALF, :])
                for p in peers:
                    rdma = pltpu.make_async_remote_copy(
                        src_ref=ctx_all.at[my, b, pl.ds(j * HALF, HALF)],
                        dst_ref=ctx_all.at[my, b, pl.ds(j * HALF, HALF)],
                        send_sem=send_sems.at[p, b, j],
                        recv_sem=recv_sems.at[my, b, j],
                        device_id=(p,),
                        device_id_type=pl.DeviceIdType.MESH,
                    )
                    rdma.start()
                    sends.append(rdma)

        cp_wo.wait()
        wo_my = wov[pl.ds(col0, D_HEADS), :].astype(jnp.bfloat16)
        ctxf = ctx_all[my, :, :, :].reshape(B * SQ, D_HEADS)
        part = jnp.dot(ctxf, wo_my,
                       preferred_element_type=jnp.float32)

        for b in range(B):
            for j in range(2):
                acc = part[b * SQ + j * HALF:b * SQ + (j + 1) * HALF, :]
                for p in peers:
                    recv = pltpu.make_async_remote_copy(
                        src_ref=ctx_all.at[p, b, pl.ds(j * HALF, HALF)],
                        dst_ref=ctx_all.at[p, b, pl.ds(j * HALF, HALF)],
                        send_sem=send_sems.at[p, b, j],
                        recv_sem=recv_sems.at[p, b, j],
                        device_id=(p,),
                        device_id_type=pl.DeviceIdType.MESH,
                    )
                    recv.wait_recv()
                    wo_p = wov[pl.ds(p * D_HEADS, D_HEADS), :].astype(
                        jnp.bfloat16)
                    acc = acc + jnp.dot(
                        ctx_all[p, b, j * HALF:(j + 1) * HALF, :], wo_p,
                        preferred_element_type=jnp.float32)
                out_ref[b, j * HALF:(j + 1) * HALF, :] = acc.astype(
                    jnp.bfloat16)

        for rdma in sends:
            rdma.wait_send()

    return pl.pallas_call(
        body,
        out_shape=jax.ShapeDtypeStruct((B, SQ, D_MODEL), jnp.bfloat16),
        in_specs=[pl.BlockSpec(memory_space=pl.ANY)] * 5,
        out_specs=pl.BlockSpec(memory_space=pltpu.VMEM),
        scratch_shapes=[
            pltpu.VMEM((B * SQ, D_MODEL), jnp.float32),
            pltpu.VMEM((D_MODEL, D_HEADS), jnp.float32),
            pltpu.VMEM((B, SKV, D_HEADS), jnp.float32),
            pltpu.VMEM((B, SKV, D_HEADS), jnp.float32),
            pltpu.VMEM((HQ_LOC * N_DEV * DH, D_MODEL), jnp.float32),
            pltpu.VMEM((N_DEV, B, SQ, D_HEADS), jnp.bfloat16),
            pltpu.SemaphoreType.DMA((5,)),
            pltpu.SemaphoreType.DMA((N_DEV, B, 2)),
            pltpu.SemaphoreType.DMA((N_DEV, B, 2)),
        ],
        compiler_params=pltpu.CompilerParams(collective_id=0),
    )(xf, wq_dev, kf, vf, Wo)
